# baseline (speedup 1.0000x reference)
import functools

import jax
import jax.numpy as jnp
from jax import lax
from jax.experimental import pallas as pl
from jax.experimental.pallas import tpu as pltpu
from jax.experimental.pallas import tpu_sc as plsc

D = 64
NBUF = 4
BU = 128
PITCH = 129


@functools.cache
def _make_sc_gather(T: int, B0: int):
    n_workers = 32
    bw = B0 // n_workers
    upt = bw // BU
    n_units = T * upt
    n_rounds = n_units // NBUF
    mesh = plsc.VectorSubcoreMesh(core_axis_name="c", subcore_axis_name="s")

    @functools.partial(
        pl.kernel,
        mesh=mesh,
        compiler_params=pltpu.CompilerParams(
            use_tc_tiling_on_sc=False, needs_layout_passes=False
        ),
        out_type=jax.ShapeDtypeStruct((T, D, B0), jnp.float32),
        scratch_types=[
            pltpu.VMEM((NBUF, BU), jnp.int32),
            pltpu.VMEM((NBUF, BU, D), jnp.float32),
            pltpu.VMEM((NBUF, D, PITCH), jnp.float32),
            pltpu.SemaphoreType.DMA((NBUF,)),
            pltpu.SemaphoreType.DMA((NBUF,)),
            pltpu.SemaphoreType.DMA((NBUF,)),
        ],
    )
    def k(w_hbm, xt_hbm, out_hbm, idx_v, gbuf, tbuf, isem, gsem, osem):
        wid = lax.axis_index("s") * 2 + lax.axis_index("c")
        col0 = wid * bw

        iota = lax.iota(jnp.int32, 16)

        def unit_tb(u):
            return u // upt, col0 + (u % upt) * BU

        def idx_load(u, slot):
            t, b0 = unit_tb(u)
            return pltpu.make_async_copy(
                xt_hbm.at[t, pl.ds(b0, BU)], idx_v.at[slot], isem.at[slot]
            )

        def gather(slot):
            return pltpu.make_async_copy(
                w_hbm.at[idx_v.at[slot]], gbuf.at[slot], gsem.at[slot]
            )

        def write(u, slot):
            t, b0 = unit_tb(u)
            return pltpu.make_async_copy(
                tbuf.at[slot, :, pl.ds(0, BU)],
                out_hbm.at[t, :, pl.ds(b0, BU)],
                osem.at[slot],
            )

        def transpose(slot):
            def bbody(g, carry):
                bb = g * 16
                for j in range(16):
                    b = bb + j
                    col = lax.broadcast(b, (16,))
                    for kk in range(4):
                        v = gbuf[slot, b, pl.ds(kk * 16, 16)]
                        plsc.store_scatter(
                            tbuf.at[slot], [iota + kk * 16, col], v
                        )
                return carry

            lax.fori_loop(0, 8, bbody, 0)

        for s in range(NBUF):
            idx_load(s, s).start()
            idx_load(s, s).wait()
            gather(s).start()

        def round_body(r, carry):
            for slot in range(NBUF):
                u = r * NBUF + slot
                gather(slot).wait()
                nxt = u + NBUF

                @pl.when(nxt < n_units)
                def _():
                    idx_load(nxt, slot).start()

                @pl.when(u >= NBUF)
                def _():
                    write(u - NBUF, slot).wait()

                transpose(slot)
                write(u, slot).start()

                @pl.when(nxt < n_units)
                def _():
                    idx_load(nxt, slot).wait()
                    gather(slot).start()

            return carry

        lax.fori_loop(0, n_rounds, round_body, 0)

        for s in range(NBUF):
            write(n_units - NBUF + s, s).wait()

    return k


def kernel(x, W):
    B0, T = x.shape
    xt = x.T.astype(jnp.int32)
    k = _make_sc_gather(T, B0)
    out = k(W, xt)
    return jnp.transpose(out, (2, 0, 1))
